# baseline (speedup 1.0000x reference)
import functools

import jax
import jax.numpy as jnp
from jax import lax
from jax.experimental import pallas as pl
from jax.experimental.pallas import tpu as pltpu
from jax.experimental.pallas import tpu_sc as plsc

_ROWS = 5184
_BLK = 1728
_GRID = _ROWS // _BLK
_SEG = 64
_SPB = _BLK // _SEG
_NSEG = _ROWS // _SEG
_KMAX = 32


def _fold(x):
    return x.reshape(_BLK // 8, 8, 128).sum(axis=0)


def _body(o_ref, l_ref, out_ref, scores_ref, segmax_ref, acc_ref):
    pid = pl.program_id(0)

    @pl.when(pid == 0)
    def _init():
        acc_ref[...] = jnp.zeros_like(acc_ref)

    o0 = o_ref[0]
    l0 = l_ref[0]
    posm = l0 > 0.5
    posf = posm.astype(jnp.float32)
    negm = l0 < -0.5

    mlogp = jnp.where(posm, -jnp.log(o0), 0.0)

    sl1 = jnp.zeros_like(o0)
    for c in range(1, 5):
        d = o_ref[c] - l_ref[c]
        ad = jnp.abs(d)
        sl1 = sl1 + jnp.where(ad < 1.0, 0.5 * d * d, ad - 0.5)

    acc_ref[0:8, :] = acc_ref[0:8, :] + _fold(mlogp)
    acc_ref[8:16, :] = acc_ref[8:16, :] + _fold(sl1 * posf)
    acc_ref[16:24, :] = acc_ref[16:24, :] + _fold(posf)
    acc_ref[24:32, :] = acc_ref[24:32, :] + _fold(negm.astype(jnp.float32))

    scores = jnp.where(negm, o0, -1.0)
    scores_ref[pl.ds(pid * _BLK, _BLK), :] = scores
    for j in range(_SPB):
        segmax_ref[pl.ds(pid * _SPB + j, 1), :] = jnp.max(
            scores[j * _SEG:(j + 1) * _SEG], axis=0, keepdims=True)

    @pl.when(pid == _GRID - 1)
    def _fin():
        logp_sum = jnp.sum(acc_ref[0:8, :])
        sl1_sum = jnp.sum(acc_ref[8:16, :])
        posc = jnp.sum(acc_ref[16:24, :])
        negc = jnp.sum(acc_ref[24:32, :])
        kf = jnp.minimum(jnp.float32(_KMAX), negc)

        rowid = lax.broadcasted_iota(jnp.int32, (_NSEG, 128), 0)

        def step(_, carry):
            rem, acc = carry
            sm = segmax_ref[...]
            m = jnp.max(sm)
            s = jnp.min(jnp.where(sm == m, rowid, _NSEG))
            seg = scores_ref[pl.ds(s * _SEG, _SEG), :]
            eq = seg == m
            cnt = jnp.sum(eq.astype(jnp.float32))
            valid = m > -0.5
            take = jnp.where(valid, jnp.minimum(cnt, rem), 0.0)
            acc = acc + take * (-jnp.log(1.0 - m))
            rem = rem - take
            newseg = jnp.where(eq, -1.0, seg)
            scores_ref[pl.ds(s * _SEG, _SEG), :] = newseg
            segmax_ref[pl.ds(s, 1), :] = jnp.max(newseg, axis=0, keepdims=True)
            return rem, acc

        _, negsum = lax.fori_loop(0, _KMAX, step, (kf, jnp.float32(0.0)))
        loss = 0.5 * logp_sum / posc + 0.5 * negsum / kf + sl1_sum / posc
        out_ref[...] = jnp.full((1, 1), loss)


_N = _ROWS * 128
_NW = 32
_CHUNK = _N // _NW


def _sc_probe(o0):
    mesh = plsc.VectorSubcoreMesh(core_axis_name="c", subcore_axis_name="s")

    @functools.partial(
        pl.kernel,
        out_type=jax.ShapeDtypeStruct((_NW * 16,), jnp.float32),
        mesh=mesh,
        scratch_types=[pltpu.VMEM((_CHUNK,), jnp.float32)],
    )
    def sck(o_hbm, out_hbm, buf):
        wid = lax.axis_index("s") * 2 + lax.axis_index("c")
        pltpu.sync_copy(o_hbm.at[pl.ds(wid * _CHUNK, _CHUNK)], buf)

        def body(i, m):
            return jnp.maximum(m, buf[pl.ds(i * 16, 16)])

        m = lax.fori_loop(0, _CHUNK // 16, body,
                          jnp.full((16,), -jnp.inf, jnp.float32))
        buf[pl.ds(0, 16)] = m
        pltpu.sync_copy(buf.at[pl.ds(0, 16)],
                        out_hbm.at[pl.ds(wid * 16, 16)])

    return sck(o0)


@jax.jit
def kernel(output, labels):
    ot = jnp.moveaxis(output, 2, 0).reshape(5, _ROWS, 128)
    lt = jnp.moveaxis(labels, 2, 0).reshape(5, _ROWS, 128)
    scmax = _sc_probe(ot[0].reshape(_N))
    out = pl.pallas_call(
        _body,
        grid=(_GRID,),
        in_specs=[pl.BlockSpec((5, _BLK, 128), lambda i: (0, i, 0)),
                  pl.BlockSpec((5, _BLK, 128), lambda i: (0, i, 0))],
        out_specs=pl.BlockSpec((1, 1), lambda i: (0, 0)),
        out_shape=jax.ShapeDtypeStruct((1, 1), jnp.float32),
        scratch_shapes=[
            pltpu.VMEM((_ROWS, 128), jnp.float32),
            pltpu.VMEM((_NSEG, 128), jnp.float32),
            pltpu.VMEM((32, 128), jnp.float32),
        ],
    )(ot, lt)
    flag = jnp.max(scmax)
    return jnp.where(flag > 1e30, flag, out[0, 0])


# speedup vs baseline: 1.4258x; 1.4258x over previous
import jax
import jax.numpy as jnp
from jax import lax
from jax.experimental import pallas as pl
from jax.experimental.pallas import tpu as pltpu

_ROWS = 5184
_BLK = 1728
_GRID = _ROWS // _BLK
_SEG = 64
_SPB = _BLK // _SEG
_NSEG = _ROWS // _SEG
_KMAX = 32


def _fold(x):
    return x.reshape(_BLK // 8, 8, 128).sum(axis=0)


def _body(o_ref, l_ref, out_ref, scores_ref, segmax_ref, acc_ref):
    pid = pl.program_id(0)

    @pl.when(pid == 0)
    def _init():
        acc_ref[...] = jnp.zeros_like(acc_ref)

    o0 = o_ref[0]
    l0 = l_ref[0]
    posm = l0 > 0.5
    posf = posm.astype(jnp.float32)
    negm = l0 < -0.5

    mlogp = jnp.where(posm, -jnp.log(o0), 0.0)

    sl1 = jnp.zeros_like(o0)
    for c in range(1, 5):
        d = o_ref[c] - l_ref[c]
        ad = jnp.abs(d)
        sl1 = sl1 + jnp.where(ad < 1.0, 0.5 * d * d, ad - 0.5)

    acc_ref[0:8, :] = acc_ref[0:8, :] + _fold(mlogp)
    acc_ref[8:16, :] = acc_ref[8:16, :] + _fold(sl1 * posf)
    acc_ref[16:24, :] = acc_ref[16:24, :] + _fold(posf)
    acc_ref[24:32, :] = acc_ref[24:32, :] + _fold(negm.astype(jnp.float32))

    scores = jnp.where(negm, o0, -1.0)
    scores_ref[pl.ds(pid * _BLK, _BLK), :] = scores
    for j in range(_SPB):
        segmax_ref[pl.ds(pid * _SPB + j, 1), :] = jnp.max(
            scores[j * _SEG:(j + 1) * _SEG], axis=0, keepdims=True)

    @pl.when(pid == _GRID - 1)
    def _fin():
        logp_sum = jnp.sum(acc_ref[0:8, :])
        sl1_sum = jnp.sum(acc_ref[8:16, :])
        posc = jnp.sum(acc_ref[16:24, :])
        negc = jnp.sum(acc_ref[24:32, :])
        kf = jnp.minimum(jnp.float32(_KMAX), negc)

        rowid = lax.broadcasted_iota(jnp.int32, (_NSEG, 128), 0)

        def step(_, carry):
            rem, acc = carry
            sm = segmax_ref[...]
            m = jnp.max(sm)
            s = jnp.min(jnp.where(sm == m, rowid, _NSEG))
            seg = scores_ref[pl.ds(s * _SEG, _SEG), :]
            eq = seg == m
            cnt = jnp.sum(eq.astype(jnp.float32))
            valid = m > -0.5
            take = jnp.where(valid, jnp.minimum(cnt, rem), 0.0)
            acc = acc + take * (-jnp.log(1.0 - m))
            rem = rem - take
            newseg = jnp.where(eq, -1.0, seg)
            scores_ref[pl.ds(s * _SEG, _SEG), :] = newseg
            segmax_ref[pl.ds(s, 1), :] = jnp.max(newseg, axis=0, keepdims=True)
            return rem, acc

        _, negsum = lax.fori_loop(0, _KMAX, step, (kf, jnp.float32(0.0)))
        loss = 0.5 * logp_sum / posc + 0.5 * negsum / kf + sl1_sum / posc
        out_ref[...] = jnp.full((1, 1), loss)


@jax.jit
def kernel(output, labels):
    ot = jnp.moveaxis(output, 2, 0).reshape(5, _ROWS, 128)
    lt = jnp.moveaxis(labels, 2, 0).reshape(5, _ROWS, 128)
    out = pl.pallas_call(
        _body,
        grid=(_GRID,),
        in_specs=[pl.BlockSpec((5, _BLK, 128), lambda i: (0, i, 0)),
                  pl.BlockSpec((5, _BLK, 128), lambda i: (0, i, 0))],
        out_specs=pl.BlockSpec((1, 1), lambda i: (0, 0)),
        out_shape=jax.ShapeDtypeStruct((1, 1), jnp.float32),
        scratch_shapes=[
            pltpu.VMEM((_ROWS, 128), jnp.float32),
            pltpu.VMEM((_NSEG, 128), jnp.float32),
            pltpu.VMEM((32, 128), jnp.float32),
        ],
    )(ot, lt)
    return out[0, 0]


# speedup vs baseline: 1.4262x; 1.0003x over previous
import jax
import jax.numpy as jnp
from jax import lax
from jax.experimental import pallas as pl
from jax.experimental.pallas import tpu as pltpu

_ROWS = 5184
_BLK = 1728
_GRID = _ROWS // _BLK
_SEG = 64
_SPB = _BLK // _SEG
_NSEG = _ROWS // _SEG
_KMAX = 32


def _fold(x):
    return x.reshape(_BLK // 8, 8, 128).sum(axis=0)


def _body(o0_ref, o1_ref, o2_ref, o3_ref, o4_ref,
          l0_ref, l1_ref, l2_ref, l3_ref, l4_ref,
          out_ref, scores_ref, segmax_ref, acc_ref):
    pid = pl.program_id(0)

    @pl.when(pid == 0)
    def _init():
        acc_ref[...] = jnp.zeros_like(acc_ref)

    o0 = o0_ref[...]
    l0 = l0_ref[...]
    posm = l0 > 0.5
    posf = posm.astype(jnp.float32)
    negm = l0 < -0.5

    mlogp = jnp.where(posm, -jnp.log(o0), 0.0)

    sl1 = jnp.zeros_like(o0)
    for o_ref, l_ref in ((o1_ref, l1_ref), (o2_ref, l2_ref),
                         (o3_ref, l3_ref), (o4_ref, l4_ref)):
        d = o_ref[...] - l_ref[...]
        ad = jnp.abs(d)
        sl1 = sl1 + jnp.where(ad < 1.0, 0.5 * d * d, ad - 0.5)

    acc_ref[0:8, :] = acc_ref[0:8, :] + _fold(mlogp)
    acc_ref[8:16, :] = acc_ref[8:16, :] + _fold(sl1 * posf)
    acc_ref[16:24, :] = acc_ref[16:24, :] + _fold(posf)
    acc_ref[24:32, :] = acc_ref[24:32, :] + _fold(negm.astype(jnp.float32))

    scores = jnp.where(negm, o0, -1.0)
    scores_ref[pl.ds(pid * _BLK, _BLK), :] = scores
    for j in range(_SPB):
        segmax_ref[pl.ds(pid * _SPB + j, 1), :] = jnp.max(
            scores[j * _SEG:(j + 1) * _SEG], axis=0, keepdims=True)

    @pl.when(pid == _GRID - 1)
    def _fin():
        logp_sum = jnp.sum(acc_ref[0:8, :])
        sl1_sum = jnp.sum(acc_ref[8:16, :])
        posc = jnp.sum(acc_ref[16:24, :])
        negc = jnp.sum(acc_ref[24:32, :])
        kf = jnp.minimum(jnp.float32(_KMAX), negc)

        rowid = lax.broadcasted_iota(jnp.int32, (_NSEG, 128), 0)

        def step(_, carry):
            rem, acc = carry
            sm = segmax_ref[...]
            m = jnp.max(sm)
            s = jnp.min(jnp.where(sm == m, rowid, _NSEG))
            seg = scores_ref[pl.ds(s * _SEG, _SEG), :]
            eq = seg == m
            cnt = jnp.sum(eq.astype(jnp.float32))
            valid = m > -0.5
            take = jnp.where(valid, jnp.minimum(cnt, rem), 0.0)
            acc = acc + take * (-jnp.log(1.0 - m))
            rem = rem - take
            newseg = jnp.where(eq, -1.0, seg)
            scores_ref[pl.ds(s * _SEG, _SEG), :] = newseg
            segmax_ref[pl.ds(s, 1), :] = jnp.max(newseg, axis=0, keepdims=True)
            return rem, acc

        _, negsum = lax.fori_loop(0, _KMAX, step, (kf, jnp.float32(0.0)))
        loss = 0.5 * logp_sum / posc + 0.5 * negsum / kf + sl1_sum / posc
        out_ref[...] = jnp.full((1, 1), loss)


@jax.jit
def kernel(output, labels):
    ot = jnp.moveaxis(output, 2, 0).reshape(5 * _ROWS, 128)
    lt = jnp.moveaxis(labels, 2, 0).reshape(5 * _ROWS, 128)

    def mkspec(c):
        return pl.BlockSpec((_BLK, 128), lambda i, c=c: (c * _GRID + i, 0))

    out = pl.pallas_call(
        _body,
        grid=(_GRID,),
        in_specs=[mkspec(c) for c in range(5)] * 2,
        out_specs=pl.BlockSpec((1, 1), lambda i: (0, 0)),
        out_shape=jax.ShapeDtypeStruct((1, 1), jnp.float32),
        scratch_shapes=[
            pltpu.VMEM((_ROWS, 128), jnp.float32),
            pltpu.VMEM((_NSEG, 128), jnp.float32),
            pltpu.VMEM((32, 128), jnp.float32),
        ],
    )(ot, ot, ot, ot, ot, lt, lt, lt, lt, lt)
    return out[0, 0]
